# Initial kernel scaffold; baseline (speedup 1.0000x reference)
import jax
import jax.numpy as jnp
from jax.experimental import pallas as pl


def kernel(features, adj_lst, W):
    raise NotImplementedError("write your pallas kernel here")


# speedup vs baseline: 1.2660x; 1.2660x over previous
import functools

import jax
import jax.numpy as jnp
from jax.experimental import pallas as pl
from jax.experimental.pallas import tpu as pltpu

KEEP = 0.5
N_NODES = 4096
DIM = 512
BLK = 512
GRID = N_NODES // BLK


def _layer_kernel(hf_ref, hb_ref, adj_ref, w_ref, out_ref, t_ref):
    @pl.when(pl.program_id(0) == 0)
    def _():
        t_ref[...] = jax.lax.dot_general(
            hf_ref[...].astype(jnp.bfloat16),
            w_ref[0].astype(jnp.bfloat16),
            (((1,), (0,)), ((), ())),
            preferred_element_type=jnp.float32,
        ).astype(jnp.bfloat16)

    prop = jax.lax.dot_general(
        adj_ref[0].astype(jnp.bfloat16),
        t_ref[...],
        (((1,), (0,)), ((), ())),
        preferred_element_type=jnp.float32,
    )
    out_ref[...] = jnp.maximum(KEEP * hb_ref[...] + (1.0 - KEEP) * prop, 0.0)


def _layer(h, adj_lst, W, i):
    return pl.pallas_call(
        _layer_kernel,
        grid=(GRID,),
        in_specs=[
            pl.BlockSpec((N_NODES, DIM), lambda g: (0, 0)),
            pl.BlockSpec((BLK, DIM), lambda g: (g, 0)),
            pl.BlockSpec((1, BLK, N_NODES), lambda g, i=i: (i, g, 0)),
            pl.BlockSpec((1, DIM, DIM), lambda g, i=i: (i, 0, 0)),
        ],
        out_specs=pl.BlockSpec((BLK, DIM), lambda g: (g, 0)),
        out_shape=jax.ShapeDtypeStruct((N_NODES, DIM), jnp.float32),
        scratch_shapes=[pltpu.VMEM((N_NODES, DIM), jnp.bfloat16)],
        compiler_params=pltpu.CompilerParams(
            dimension_semantics=("arbitrary",),
        ),
    )(h, h, adj_lst, W)


@jax.jit
def kernel(features, adj_lst, W):
    h = features
    for i in range(adj_lst.shape[0]):
        h = _layer(h, adj_lst, W, i)
    return h


# speedup vs baseline: 1.6606x; 1.3117x over previous
import jax
import jax.numpy as jnp
from jax.experimental import pallas as pl
from jax.experimental.pallas import tpu as pltpu

KEEP = 0.5
N_NODES = 4096
DIM = 512
BLK = 512
GRID = N_NODES // BLK
N_LAYERS = 4


def _bf16_mm(a, b):
    return jax.lax.dot_general(
        a.astype(jnp.bfloat16),
        b.astype(jnp.bfloat16),
        (((1,), (0,)), ((), ())),
        preferred_element_type=jnp.float32,
    )


def _fused_kernel(f_ref, adj_ref, w_ref, out_ref, h_ref, t_ref):
    l = pl.program_id(0)
    g = pl.program_id(1)
    cur = jax.lax.rem(l, 2)
    nxt = 1 - cur

    @pl.when((l == 0) & (g == 0))
    def _():
        t_ref[0] = _bf16_mm(f_ref[...], w_ref[0]).astype(jnp.bfloat16)

    rows = pl.ds(g * BLK, BLK)
    h_in = jnp.where(l == 0, f_ref[rows, :], h_ref[rows, :])
    prop = _bf16_mm(adj_ref[0], t_ref[cur])
    new_h = jnp.maximum(KEEP * h_in + (1.0 - KEEP) * prop, 0.0)
    h_ref[rows, :] = new_h

    @pl.when(l < N_LAYERS - 1)
    def _():
        t_ref[nxt, rows, :] = _bf16_mm(new_h, w_ref[l + 1]).astype(jnp.bfloat16)

    @pl.when(l == N_LAYERS - 1)
    def _():
        out_ref[...] = new_h


@jax.jit
def kernel(features, adj_lst, W):
    out = pl.pallas_call(
        _fused_kernel,
        grid=(N_LAYERS, GRID),
        in_specs=[
            pl.BlockSpec((N_NODES, DIM), lambda l, g: (0, 0)),
            pl.BlockSpec((1, BLK, N_NODES), lambda l, g: (l, g, 0)),
            pl.BlockSpec((N_LAYERS, DIM, DIM), lambda l, g: (0, 0, 0)),
        ],
        out_specs=pl.BlockSpec(
            (BLK, DIM),
            lambda l, g: (jnp.where(l == N_LAYERS - 1, g, 0), 0),
        ),
        out_shape=jax.ShapeDtypeStruct((N_NODES, DIM), jnp.float32),
        scratch_shapes=[
            pltpu.VMEM((N_NODES, DIM), jnp.float32),
            pltpu.VMEM((2, N_NODES, DIM), jnp.bfloat16),
        ],
        compiler_params=pltpu.CompilerParams(
            dimension_semantics=("arbitrary", "arbitrary"),
        ),
    )(features, adj_lst, W)
    return out
